# Initial kernel scaffold; baseline (speedup 1.0000x reference)
import jax
import jax.numpy as jnp
from jax.experimental import pallas as pl


def kernel(inputs, embeddings):
    raise NotImplementedError("write your pallas kernel here")


# speedup vs baseline: 2.1889x; 2.1889x over previous
import functools

import jax
import jax.numpy as jnp
from jax import lax
from jax.experimental import pallas as pl

LATENT = 100
NUM_EMB = 100


def _vq_block(x_ref, e_ref, o_ref):
    x = x_ref[...]
    e = e_ref[...]
    dot = lax.dot_general(x, e, (((1,), (1,)), ((), ())),
                          preferred_element_type=jnp.float32)
    x_sq = jnp.sum(x * x, axis=1, keepdims=True)
    e_sq = jnp.sum(e * e, axis=1)[None, :]
    d2 = x_sq - 2.0 * dot + e_sq
    dists = jnp.sqrt(jnp.maximum(d2, 0.0))
    dmin = jnp.min(dists, axis=1, keepdims=True)
    iota_k = lax.broadcasted_iota(jnp.int32, dists.shape, 1)
    idx = jnp.min(jnp.where(dists == dmin, iota_k, NUM_EMB), axis=1)
    oh = (iota_k == idx[:, None]).astype(jnp.float32)
    o_ref[...] = lax.dot_general(oh, e, (((1,), (0,)), ((), ())),
                                 preferred_element_type=jnp.float32,
                                 precision=lax.Precision.HIGHEST)


@functools.partial(jax.jit, static_argnames=("interpret",))
def _vq(inputs, embeddings, interpret=False):
    shape = inputs.shape
    flat = jnp.reshape(inputs, (-1, LATENT))
    n = flat.shape[0]
    bn = 2048
    out = pl.pallas_call(
        _vq_block,
        grid=(n // bn,),
        in_specs=[
            pl.BlockSpec((bn, LATENT), lambda i: (i, 0)),
            pl.BlockSpec((NUM_EMB, LATENT), lambda i: (0, 0)),
        ],
        out_specs=pl.BlockSpec((bn, LATENT), lambda i: (i, 0)),
        out_shape=jax.ShapeDtypeStruct((n, LATENT), jnp.float32),
        interpret=interpret,
    )(flat, embeddings)
    return jnp.reshape(out, shape)


def kernel(inputs, embeddings):
    return _vq(inputs, embeddings)


# speedup vs baseline: 2.3563x; 1.0765x over previous
import functools

import jax
import jax.numpy as jnp
from jax import lax
from jax.experimental import pallas as pl

LATENT = 100
NUM_EMB = 100


def _vq_block(x_ref, e_ref, xsq_ref, esq_ref, o_ref):
    xb = x_ref[...]
    eb = e_ref[...]
    dt = lax.dot_general(eb, xb, (((1,), (1,)), ((), ())),
                         preferred_element_type=jnp.float32)
    x_sq = jnp.reshape(xsq_ref[...], (1, -1))
    e_sq = esq_ref[...]
    d2 = (x_sq - 2.0 * dt) + e_sq
    dists = jnp.sqrt(jnp.maximum(d2, 0.0))
    dmin = jnp.min(dists, axis=0, keepdims=True)
    iota_k = lax.broadcasted_iota(jnp.int32, dists.shape, 0)
    idx = jnp.min(jnp.where(dists == dmin, iota_k, NUM_EMB), axis=0)
    oh = (iota_k == idx[None, :]).astype(jnp.float32)
    o_ref[...] = lax.dot_general(oh, eb, (((0,), (0,)), ((), ())),
                                 preferred_element_type=jnp.float32,
                                 precision=lax.Precision.HIGHEST)


@functools.partial(jax.jit, static_argnames=("interpret",))
def _vq(inputs, embeddings, interpret=False):
    shape = inputs.shape
    flat = jnp.reshape(inputs, (-1, LATENT))
    n = flat.shape[0]
    bn = 2048
    nb = n // bn
    x_sq = jnp.reshape(jnp.sum(flat * flat, axis=1), (nb, 1, bn))
    e_sq = jnp.reshape(jnp.sum(embeddings * embeddings, axis=1), (NUM_EMB, 1))
    out = pl.pallas_call(
        _vq_block,
        grid=(nb,),
        in_specs=[
            pl.BlockSpec((bn, LATENT), lambda i: (i, 0)),
            pl.BlockSpec((NUM_EMB, LATENT), lambda i: (0, 0)),
            pl.BlockSpec((1, 1, bn), lambda i: (i, 0, 0)),
            pl.BlockSpec((NUM_EMB, 1), lambda i: (0, 0)),
        ],
        out_specs=pl.BlockSpec((bn, LATENT), lambda i: (i, 0)),
        out_shape=jax.ShapeDtypeStruct((n, LATENT), jnp.float32),
        interpret=interpret,
    )(flat, embeddings, x_sq, e_sq)
    return jnp.reshape(out, shape)


def kernel(inputs, embeddings):
    return _vq(inputs, embeddings)
